# Initial kernel scaffold; baseline (speedup 1.0000x reference)
import jax
import jax.numpy as jnp
from jax.experimental import pallas as pl


def kernel(x_num, x_cat, weight, bias, emb_table, category_offsets):
    raise NotImplementedError("write your pallas kernel here")


# speedup vs baseline: 1.0169x; 1.0169x over previous
import jax
import jax.numpy as jnp
from jax import lax
from jax.experimental import pallas as pl
from jax.experimental.pallas import tpu as pltpu, tpu_sc as plsc

B = 16384
D_NUM = 13
N_CAT = 26
D_TOKEN = 16
N_TOK = D_NUM + N_CAT
OUT_W = N_TOK * D_TOKEN
NC, NS = 2, 16
NW = NC * NS
ROWS_PER_W = B // NW
R = 64
NCHUNK = ROWS_PER_W // R
IDX_PER_CHUNK = R * N_CAT
G = 128
NG = IDX_PER_CHUNK // G


def _body(xnum_hbm, xcat_hbm, offt_hbm, w_hbm, b_hbm, table_hbm, out_hbm,
          idx_v, off_v, w_v, b_v, xnum_v, staged_v, out_v, sem):
    wid = lax.axis_index("s") * NC + lax.axis_index("c")
    row0 = wid * ROWS_PER_W

    pltpu.sync_copy(w_hbm, w_v)
    pltpu.sync_copy(b_hbm, b_v)
    pltpu.sync_copy(offt_hbm, off_v)

    def chunk_body(k, carry):
        b0 = row0 + k * R
        q = row0 // R + k
        pltpu.sync_copy(xcat_hbm.at[q], idx_v)
        pltpu.sync_copy(xnum_hbm.at[pl.ds(b0, R)], xnum_v)

        for j in range(NG):
            for i in range(G // 16):
                s = pl.ds(i * 16, 16)
                idx_v[j, s] = idx_v[j, s] + off_v[j, s]

        cps = [
            pltpu.async_copy(table_hbm.at[idx_v.at[j]],
                             staged_v.at[pl.ds(j * G, G)], sem)
            for j in range(NG)
        ]

        def num_body(r, c):
            v = xnum_v[r, :]
            for d in range(D_NUM):
                out_v[r, pl.ds(d * D_TOKEN, D_TOKEN)] = (
                    w_v[d, :] * v[d] + b_v[d, :])
            return c

        lax.fori_loop(0, R, num_body, 0, unroll=2)

        for cp in cps:
            cp.wait()

        def cat_body(r, c):
            base = r * N_CAT
            for cc in range(N_CAT):
                out_v[r, pl.ds((D_NUM + cc) * D_TOKEN, D_TOKEN)] = (
                    staged_v[base + cc, :] + b_v[D_NUM + cc, :])
            return c

        lax.fori_loop(0, R, cat_body, 0, unroll=2)

        pltpu.sync_copy(out_v, out_hbm.at[pl.ds(b0, R)])
        return carry

    lax.fori_loop(0, NCHUNK, chunk_body, 0)


@jax.jit
def _tokenizer(x_num, xcat2d, off_tile, weight, bias, emb_table):
    mesh = plsc.VectorSubcoreMesh(core_axis_name="c", subcore_axis_name="s",
                                  num_cores=NC, num_subcores=NS)
    f = pl.kernel(
        _body,
        out_type=jax.ShapeDtypeStruct((B, OUT_W), jnp.float32),
        mesh=mesh,
        compiler_params=pltpu.CompilerParams(use_tc_tiling_on_sc=False),
        scratch_types=[
            pltpu.VMEM((NG, G), jnp.int32),
            pltpu.VMEM((NG, G), jnp.int32),
            pltpu.VMEM((D_NUM, D_TOKEN), jnp.float32),
            pltpu.VMEM((N_TOK, D_TOKEN), jnp.float32),
            pltpu.VMEM((R, D_TOKEN), jnp.float32),
            pltpu.VMEM((IDX_PER_CHUNK, D_TOKEN), jnp.float32),
            pltpu.VMEM((R, OUT_W), jnp.float32),
            pltpu.SemaphoreType.DMA,
        ],
    )
    return f(x_num, xcat2d, off_tile, weight, bias, emb_table)


def kernel(x_num, x_cat, weight, bias, emb_table, category_offsets):
    x_num16 = jnp.pad(x_num, ((0, 0), (0, D_TOKEN - D_NUM)))
    xcat2d = x_cat.reshape(B // R, NG, G)
    off_tile = jnp.tile(category_offsets, R).reshape(NG, G)
    return _tokenizer(x_num16, xcat2d, off_tile, weight, bias, emb_table)


# speedup vs baseline: 1.0261x; 1.0090x over previous
import jax
import jax.numpy as jnp
from jax import lax
from jax.experimental import pallas as pl
from jax.experimental.pallas import tpu as pltpu, tpu_sc as plsc

B = 16384
D_NUM = 13
N_CAT = 26
D_TOKEN = 16
N_TOK = D_NUM + N_CAT
OUT_W = N_TOK * D_TOKEN
NUM_W = D_NUM * D_TOKEN
CAT_W = N_CAT * D_TOKEN
NC, NS = 2, 16
NW = NC * NS
ROWS_PER_W = B // NW
R = 64
NCHUNK = ROWS_PER_W // R
IDX_PER_CHUNK = R * N_CAT
G = 128
NG = IDX_PER_CHUNK // G
BB = 128
NBB = B // BB
ST_ROWS = B * N_CAT


def _sc_body(xcat_hbm, offt_hbm, table_hbm, st_hbm,
             idx_v, off_v, staged_v, sem):
    wid = lax.axis_index("s") * NC + lax.axis_index("c")
    row0 = wid * ROWS_PER_W

    pltpu.sync_copy(offt_hbm, off_v)

    def chunk_body(k, carry):
        b0 = row0 + k * R
        q = row0 // R + k
        pltpu.sync_copy(xcat_hbm.at[q], idx_v)

        for j in range(NG):
            for i in range(G // 16):
                s = pl.ds(i * 16, 16)
                idx_v[j, s] = idx_v[j, s] + off_v[j, s]

        cps = [
            pltpu.async_copy(table_hbm.at[idx_v.at[j]],
                             staged_v.at[pl.ds(j * G, G)], sem)
            for j in range(NG)
        ]
        for cp in cps:
            cp.wait()

        pltpu.sync_copy(staged_v, st_hbm.at[pl.ds(b0 * N_CAT, IDX_PER_CHUNK)])
        return carry

    lax.fori_loop(0, NCHUNK, chunk_body, 0)


def _tc_body(st_ref, xr_ref, wf_ref, bn_ref, bc_ref, out_ref):
    num = xr_ref[...] * wf_ref[...] + bn_ref[...]
    cat = st_ref[...] + bc_ref[...]
    out_ref[...] = jnp.concatenate([num, cat], axis=1)


def _sc_gather(xcat3d, off_tile, emb_table):
    mesh = plsc.VectorSubcoreMesh(core_axis_name="c", subcore_axis_name="s",
                                  num_cores=NC, num_subcores=NS)
    f = pl.kernel(
        _sc_body,
        out_type=jax.ShapeDtypeStruct((ST_ROWS, D_TOKEN), jnp.float32),
        mesh=mesh,
        compiler_params=pltpu.CompilerParams(use_tc_tiling_on_sc=False),
        scratch_types=[
            pltpu.VMEM((NG, G), jnp.int32),
            pltpu.VMEM((NG, G), jnp.int32),
            pltpu.VMEM((IDX_PER_CHUNK, D_TOKEN), jnp.float32),
            pltpu.SemaphoreType.DMA,
        ],
    )
    return f(xcat3d, off_tile, emb_table)


def _tc_assemble(st2, xr, wf, bn, bc):
    return pl.pallas_call(
        _tc_body,
        grid=(NBB,),
        in_specs=[
            pl.BlockSpec((BB, CAT_W), lambda i: (i, 0)),
            pl.BlockSpec((BB, NUM_W), lambda i: (i, 0)),
            pl.BlockSpec((1, NUM_W), lambda i: (0, 0)),
            pl.BlockSpec((1, NUM_W), lambda i: (0, 0)),
            pl.BlockSpec((1, CAT_W), lambda i: (0, 0)),
        ],
        out_specs=pl.BlockSpec((BB, OUT_W), lambda i: (i, 0)),
        out_shape=jax.ShapeDtypeStruct((B, OUT_W), jnp.float32),
    )(st2, xr, wf, bn, bc)


def kernel(x_num, x_cat, weight, bias, emb_table, category_offsets):
    xcat3d = x_cat.reshape(B // R, NG, G)
    off_tile = jnp.tile(category_offsets, R).reshape(NG, G)
    st = _sc_gather(xcat3d, off_tile, emb_table)
    st2 = st.reshape(B, CAT_W)
    xr = jnp.repeat(x_num, D_TOKEN, axis=1)
    wf = weight.reshape(1, NUM_W)
    bn = bias[:D_NUM].reshape(1, NUM_W)
    bc = bias[D_NUM:].reshape(1, CAT_W)
    return _tc_assemble(st2, xr, wf, bn, bc)
